# Initial kernel scaffold; baseline (speedup 1.0000x reference)
import jax
import jax.numpy as jnp
from jax.experimental import pallas as pl


def kernel(x, hex_weights, W_in, b_in, W_out, b_out, gamma, beta):
    raise NotImplementedError("write your pallas kernel here")


# speedup vs baseline: 4.7544x; 4.7544x over previous
import jax
import jax.numpy as jnp
from jax.experimental import pallas as pl
from jax.experimental.pallas import tpu as pltpu

_B, _T, _D, _F, _E, _TOPK = 16, 4, 768, 1536, 64, 2
_N = _B * _T


def _routing_matrix(hexw):
    col = jax.lax.broadcasted_iota(jnp.int32, (_N, _E), 1)
    m1 = jnp.max(hexw, axis=1, keepdims=True)
    cand1 = jnp.where(hexw == m1, col, _E)
    i1 = jnp.min(cand1, axis=1, keepdims=True)
    mask1 = col == i1
    hex2 = jnp.where(mask1, -jnp.inf, hexw)
    m2 = jnp.max(hex2, axis=1, keepdims=True)
    cand2 = jnp.where(hex2 == m2, col, _E)
    i2 = jnp.min(cand2, axis=1, keepdims=True)
    mask2 = col == i2
    denom = m1 + m2 + 1e-8
    return (jnp.where(mask1, m1 / denom, 0.0)
            + jnp.where(mask2, m2 / denom, 0.0))


def _fwd(x_ref, hex_ref, wi_ref, bi_ref, wo_ref, bo_ref, g_ref, bt_ref,
         out_ref, xn_ref, r_ref):
    e = pl.program_id(0)

    @pl.when(e == 0)
    def _prologue():
        xx = x_ref[...]
        mu = jnp.mean(xx, axis=1, keepdims=True)
        var = jnp.mean((xx - mu) ** 2, axis=1, keepdims=True)
        xn_ref[...] = ((xx - mu) * jax.lax.rsqrt(var + 1e-5) * g_ref[...]
                       + bt_ref[...])
        r_ref[...] = _routing_matrix(hex_ref[...])
        out_ref[...] = jnp.zeros_like(out_ref)

    xn = xn_ref[...]
    h = jax.lax.dot_general(xn, wi_ref[0], (((1,), (1,)), ((), ())),
                            preferred_element_type=jnp.float32)
    h = h + bi_ref[0]
    h = h * jax.nn.sigmoid(h)
    o = jax.lax.dot_general(h, wo_ref[0], (((1,), (1,)), ((), ())),
                            preferred_element_type=jnp.float32)
    o = o + bo_ref[0]
    col = jax.lax.broadcasted_iota(jnp.int32, (_N, _E), 1)
    r_e = jnp.sum(jnp.where(col == e, r_ref[...], 0.0), axis=1,
                  keepdims=True)
    out_ref[...] += r_e * o


def kernel(x, hex_weights, W_in, b_in, W_out, b_out, gamma, beta):
    xf = x.reshape(_N, _D)
    hexf = hex_weights.reshape(_N, _E)
    bi3 = b_in.reshape(_E, 1, _F)
    bo3 = b_out.reshape(_E, 1, _D)
    g2 = gamma.reshape(1, _D)
    bt2 = beta.reshape(1, _D)

    out = pl.pallas_call(
        _fwd,
        grid=(_E,),
        in_specs=[
            pl.BlockSpec((_N, _D), lambda e: (0, 0)),
            pl.BlockSpec((_N, _E), lambda e: (0, 0)),
            pl.BlockSpec((1, _F, _D), lambda e: (e, 0, 0)),
            pl.BlockSpec((1, 1, _F), lambda e: (e, 0, 0)),
            pl.BlockSpec((1, _D, _F), lambda e: (e, 0, 0)),
            pl.BlockSpec((1, 1, _D), lambda e: (e, 0, 0)),
            pl.BlockSpec((1, _D), lambda e: (0, 0)),
            pl.BlockSpec((1, _D), lambda e: (0, 0)),
        ],
        out_specs=pl.BlockSpec((_N, _D), lambda e: (0, 0)),
        out_shape=jax.ShapeDtypeStruct((_N, _D), jnp.float32),
        scratch_shapes=[
            pltpu.VMEM((_N, _D), jnp.float32),
            pltpu.VMEM((_N, _E), jnp.float32),
        ],
        compiler_params=pltpu.CompilerParams(
            dimension_semantics=("arbitrary",),
        ),
    )(xf, hexf, W_in, bi3, W_out, bo3, g2, bt2)
    return out.reshape(_B, _T, _D)


# speedup vs baseline: 4.9715x; 1.0457x over previous
import functools

import jax
import jax.numpy as jnp
from jax.experimental import pallas as pl
from jax.experimental.pallas import tpu as pltpu
from jax.experimental.pallas import tpu_sc as plsc

_B, _T, _D, _F, _E, _TOPK = 16, 4, 768, 1536, 64, 2
_N = _B * _T
_L = 16
_EC = _E // _L


def _sc_router_body(hex_hbm, used_hbm, hexv, histv, usedv):
    cid = jax.lax.axis_index("c")
    sid = jax.lax.axis_index("s")

    @pl.when((cid == 0) & (sid == 0))
    def _tile0():
        pltpu.sync_copy(hex_hbm, hexv)
        lane = jax.lax.iota(jnp.int32, _L)
        ones = jnp.ones((_L,), jnp.float32)
        zeros = jnp.zeros((_L,), jnp.float32)
        for c in range(_EC):
            histv[pl.ds(_L * c, _L)] = zeros

        def row_body(r, carry):
            chunks = [hexv[r, pl.ds(_L * c, _L)] for c in range(_EC)]
            vmax = chunks[0]
            for ch in chunks[1:]:
                vmax = jnp.maximum(vmax, ch)
            m1 = jnp.max(vmax)
            cand = None
            for c, ch in enumerate(chunks):
                cc = jnp.where(ch == m1, lane + _L * c, jnp.int32(4 * _E))
                cand = cc if cand is None else jnp.minimum(cand, cc)
            i1 = jnp.min(cand)
            vmax2 = None
            chunks2 = []
            for c, ch in enumerate(chunks):
                ch2 = jnp.where(lane + _L * c == i1, -jnp.inf, ch)
                chunks2.append(ch2)
                vmax2 = ch2 if vmax2 is None else jnp.maximum(vmax2, ch2)
            m2 = jnp.max(vmax2)
            cand2 = None
            for c, ch2 in enumerate(chunks2):
                cc = jnp.where(ch2 == m2, lane + _L * c, jnp.int32(4 * _E))
                cand2 = cc if cand2 is None else jnp.minimum(cand2, cc)
            i2 = jnp.min(cand2)
            iv = jnp.where(lane == 0, i1, i2)
            plsc.store_scatter(histv, [iv], ones, mask=lane < 2)
            return carry

        jax.lax.fori_loop(0, _N, row_body, jnp.int32(0))

        lastu = jnp.int32(-1)
        for c in range(_EC):
            pres = histv[pl.ds(_L * c, _L)] > 0.0
            lastu = jnp.maximum(
                lastu, jnp.max(jnp.where(pres, lane + _L * c, jnp.int32(-1))))
        lastv = jnp.broadcast_to(lastu, (_L,))
        for c in range(_EC):
            usedv[pl.ds(_L * c, _L)] = lastv

        off = jnp.int32(0)
        for c in range(_EC):
            pres = histv[pl.ds(_L * c, _L)] > 0.0
            presi = pres.astype(jnp.int32)
            pos = off + plsc.cumsum(presi) - 1
            plsc.store_scatter(usedv, [pos], lane + _L * c, mask=pres)
            off = off + jnp.sum(presi)

        pltpu.sync_copy(usedv, used_hbm)


def _sc_router(hexf):
    fn = pl.kernel(
        _sc_router_body,
        out_type=jax.ShapeDtypeStruct((_E,), jnp.int32),
        mesh=plsc.VectorSubcoreMesh(core_axis_name="c", subcore_axis_name="s"),
        scratch_types=[
            pltpu.VMEM((_N, _E), jnp.float32),
            pltpu.VMEM((_E,), jnp.float32),
            pltpu.VMEM((_E,), jnp.int32),
        ],
        compiler_params=pltpu.CompilerParams(needs_layout_passes=False),
    )
    return fn(hexf)


def _routing_matrix(hexw):
    col = jax.lax.broadcasted_iota(jnp.int32, (_N, _E), 1)
    m1 = jnp.max(hexw, axis=1, keepdims=True)
    cand1 = jnp.where(hexw == m1, col, _E)
    i1 = jnp.min(cand1, axis=1, keepdims=True)
    mask1 = col == i1
    hex2 = jnp.where(mask1, -jnp.inf, hexw)
    m2 = jnp.max(hex2, axis=1, keepdims=True)
    cand2 = jnp.where(hex2 == m2, col, _E)
    i2 = jnp.min(cand2, axis=1, keepdims=True)
    mask2 = col == i2
    denom = m1 + m2 + 1e-8
    return (jnp.where(mask1, m1 / denom, 0.0)
            + jnp.where(mask2, m2 / denom, 0.0))


def _fwd(s_ref, x_ref, hex_ref, wi_ref, bi_ref, wo_ref, bo_ref, g_ref,
         bt_ref, out_ref, xn_ref, r_ref):
    i = pl.program_id(0)
    e = s_ref[i]
    prev = s_ref[jnp.maximum(i - 1, 0)]
    gate = jnp.logical_or(i == 0, e != prev)

    @pl.when(i == 0)
    def _prologue():
        xx = x_ref[...]
        mu = jnp.mean(xx, axis=1, keepdims=True)
        var = jnp.mean((xx - mu) ** 2, axis=1, keepdims=True)
        xn_ref[...] = ((xx - mu) * jax.lax.rsqrt(var + 1e-5) * g_ref[...]
                       + bt_ref[...])
        r_ref[...] = _routing_matrix(hex_ref[...])
        out_ref[...] = jnp.zeros_like(out_ref)

    @pl.when(gate)
    def _expert():
        xn = xn_ref[...]
        h = jax.lax.dot_general(xn, wi_ref[0], (((1,), (1,)), ((), ())),
                                preferred_element_type=jnp.float32)
        h = h + bi_ref[0]
        h = h * jax.nn.sigmoid(h)
        o = jax.lax.dot_general(h, wo_ref[0], (((1,), (1,)), ((), ())),
                                preferred_element_type=jnp.float32)
        o = o + bo_ref[0]
        col = jax.lax.broadcasted_iota(jnp.int32, (_N, _E), 1)
        r_e = jnp.sum(jnp.where(col == e, r_ref[...], 0.0), axis=1,
                      keepdims=True)
        out_ref[...] += r_e * o


def kernel(x, hex_weights, W_in, b_in, W_out, b_out, gamma, beta):
    xf = x.reshape(_N, _D)
    hexf = hex_weights.reshape(_N, _E)
    bi3 = b_in.reshape(_E, 1, _F)
    bo3 = b_out.reshape(_E, 1, _D)
    g2 = gamma.reshape(1, _D)
    bt2 = beta.reshape(1, _D)

    used = _sc_router(hexf)

    grid_spec = pltpu.PrefetchScalarGridSpec(
        num_scalar_prefetch=1,
        grid=(_E,),
        in_specs=[
            pl.BlockSpec((_N, _D), lambda i, s: (0, 0)),
            pl.BlockSpec((_N, _E), lambda i, s: (0, 0)),
            pl.BlockSpec((1, _F, _D), lambda i, s: (s[i], 0, 0)),
            pl.BlockSpec((1, 1, _F), lambda i, s: (s[i], 0, 0)),
            pl.BlockSpec((1, _D, _F), lambda i, s: (s[i], 0, 0)),
            pl.BlockSpec((1, 1, _D), lambda i, s: (s[i], 0, 0)),
            pl.BlockSpec((1, _D), lambda i, s: (0, 0)),
            pl.BlockSpec((1, _D), lambda i, s: (0, 0)),
        ],
        out_specs=pl.BlockSpec((_N, _D), lambda i, s: (0, 0)),
        scratch_shapes=[
            pltpu.VMEM((_N, _D), jnp.float32),
            pltpu.VMEM((_N, _E), jnp.float32),
        ],
    )
    out = pl.pallas_call(
        _fwd,
        grid_spec=grid_spec,
        out_shape=jax.ShapeDtypeStruct((_N, _D), jnp.float32),
        compiler_params=pltpu.CompilerParams(
            dimension_semantics=("arbitrary",),
        ),
    )(used, xf, hexf, W_in, bi3, W_out, bo3, g2, bt2)
    return out.reshape(_B, _T, _D)


# speedup vs baseline: 5.3756x; 1.0813x over previous
import functools

import jax
import jax.numpy as jnp
from jax.experimental import pallas as pl
from jax.experimental.pallas import tpu as pltpu
from jax.experimental.pallas import tpu_sc as plsc

_B, _T, _D, _F, _E, _TOPK = 16, 4, 768, 1536, 64, 2
_N = _B * _T
_L = 16
_EC = _E // _L


_RPS = _N // 16


def _top2_row(chunks, lane):
    vmax = chunks[0]
    for ch in chunks[1:]:
        vmax = jnp.maximum(vmax, ch)
    m1 = jnp.max(vmax)
    cand = None
    for c, ch in enumerate(chunks):
        cc = jnp.where(ch == m1, lane + _L * c, jnp.int32(4 * _E))
        cand = cc if cand is None else jnp.minimum(cand, cc)
    i1 = jnp.min(cand)
    vmax2 = None
    chunks2 = []
    for c, ch in enumerate(chunks):
        ch2 = jnp.where(lane + _L * c == i1, -jnp.inf, ch)
        chunks2.append(ch2)
        vmax2 = ch2 if vmax2 is None else jnp.maximum(vmax2, ch2)
    m2 = jnp.max(vmax2)
    cand2 = None
    for c, ch2 in enumerate(chunks2):
        cc = jnp.where(ch2 == m2, lane + _L * c, jnp.int32(4 * _E))
        cand2 = cc if cand2 is None else jnp.minimum(cand2, cc)
    i2 = jnp.min(cand2)
    return i1, i2


def _sc_router_body(hex_hbm, used_hbm, hexv, histv, usedv, idxv, shared):
    cid = jax.lax.axis_index("c")
    sid = jax.lax.axis_index("s")
    lane = jax.lax.iota(jnp.int32, _L)

    @pl.when(cid == 0)
    def _per_subcore():
        pltpu.sync_copy(hex_hbm.at[pl.ds(sid * _RPS, _RPS)], hexv)
        pairs = []
        for r in range(_RPS):
            chunks = [hexv[r, pl.ds(_L * c, _L)] for c in range(_EC)]
            pairs.extend(_top2_row(chunks, lane))
        iv = jnp.broadcast_to(pairs[0], (_L,))
        for p, val in enumerate(pairs[1:]):
            iv = jnp.where(lane == p + 1, val, iv)
        idxv[...] = iv
        pltpu.sync_copy(idxv, shared.at[sid])

    plsc.subcore_barrier()

    @pl.when((cid == 0) & (sid == 0))
    def _tile0():
        ones = jnp.ones((_L,), jnp.float32)
        zeros = jnp.zeros((_L,), jnp.float32)
        for c in range(_EC):
            histv[pl.ds(_L * c, _L)] = zeros
        for j in range(16):
            pltpu.sync_copy(shared.at[j], idxv)
            plsc.store_scatter(histv, [idxv[...]], ones)

        lastu = jnp.int32(-1)
        for c in range(_EC):
            pres = histv[pl.ds(_L * c, _L)] > 0.0
            lastu = jnp.maximum(
                lastu, jnp.max(jnp.where(pres, lane + _L * c, jnp.int32(-1))))
        lastv = jnp.broadcast_to(lastu, (_L,))
        for c in range(_EC):
            usedv[pl.ds(_L * c, _L)] = lastv

        off = jnp.int32(0)
        for c in range(_EC):
            pres = histv[pl.ds(_L * c, _L)] > 0.0
            presi = pres.astype(jnp.int32)
            pos = off + plsc.cumsum(presi) - 1
            plsc.store_scatter(usedv, [pos], lane + _L * c, mask=pres)
            off = off + jnp.sum(presi)

        pltpu.sync_copy(usedv, used_hbm)


def _sc_router(hexf):
    fn = pl.kernel(
        _sc_router_body,
        out_type=jax.ShapeDtypeStruct((_E,), jnp.int32),
        mesh=plsc.VectorSubcoreMesh(core_axis_name="c", subcore_axis_name="s"),
        scratch_types=[
            pltpu.VMEM((_RPS, _E), jnp.float32),
            pltpu.VMEM((_E,), jnp.float32),
            pltpu.VMEM((_E,), jnp.int32),
            pltpu.VMEM((_L,), jnp.int32),
            pltpu.VMEM_SHARED((16, _L), jnp.int32),
        ],
        compiler_params=pltpu.CompilerParams(needs_layout_passes=False),
    )
    return fn(hexf)


def _routing_matrix(hexw):
    col = jax.lax.broadcasted_iota(jnp.int32, (_N, _E), 1)
    m1 = jnp.max(hexw, axis=1, keepdims=True)
    cand1 = jnp.where(hexw == m1, col, _E)
    i1 = jnp.min(cand1, axis=1, keepdims=True)
    mask1 = col == i1
    hex2 = jnp.where(mask1, -jnp.inf, hexw)
    m2 = jnp.max(hex2, axis=1, keepdims=True)
    cand2 = jnp.where(hex2 == m2, col, _E)
    i2 = jnp.min(cand2, axis=1, keepdims=True)
    mask2 = col == i2
    denom = m1 + m2 + 1e-8
    return (jnp.where(mask1, m1 / denom, 0.0)
            + jnp.where(mask2, m2 / denom, 0.0))


def _fwd(s_ref, x_ref, hex_ref, wi_ref, bi_ref, wo_ref, bo_ref, g_ref,
         bt_ref, out_ref, xn_ref, r_ref):
    i = pl.program_id(0)
    e = s_ref[i]
    prev = s_ref[jnp.maximum(i - 1, 0)]
    gate = jnp.logical_or(i == 0, e != prev)

    @pl.when(i == 0)
    def _prologue():
        xx = x_ref[...]
        mu = jnp.mean(xx, axis=1, keepdims=True)
        var = jnp.mean((xx - mu) ** 2, axis=1, keepdims=True)
        xn_ref[...] = ((xx - mu) * jax.lax.rsqrt(var + 1e-5) * g_ref[...]
                       + bt_ref[...])
        r_ref[...] = _routing_matrix(hex_ref[...])
        out_ref[...] = jnp.zeros_like(out_ref)

    @pl.when(gate)
    def _expert():
        xn = xn_ref[...]
        h = jax.lax.dot_general(xn, wi_ref[0], (((1,), (1,)), ((), ())),
                                preferred_element_type=jnp.float32)
        h = h + bi_ref[0]
        h = h * jax.nn.sigmoid(h)
        o = jax.lax.dot_general(h, wo_ref[0], (((1,), (1,)), ((), ())),
                                preferred_element_type=jnp.float32)
        o = o + bo_ref[0]
        col = jax.lax.broadcasted_iota(jnp.int32, (_N, _E), 1)
        r_e = jnp.sum(jnp.where(col == e, r_ref[...], 0.0), axis=1,
                      keepdims=True)
        out_ref[...] += r_e * o


def kernel(x, hex_weights, W_in, b_in, W_out, b_out, gamma, beta):
    xf = x.reshape(_N, _D)
    hexf = hex_weights.reshape(_N, _E)
    bi3 = b_in.reshape(_E, 1, _F)
    bo3 = b_out.reshape(_E, 1, _D)
    g2 = gamma.reshape(1, _D)
    bt2 = beta.reshape(1, _D)

    used = _sc_router(hexf)

    grid_spec = pltpu.PrefetchScalarGridSpec(
        num_scalar_prefetch=1,
        grid=(_E,),
        in_specs=[
            pl.BlockSpec((_N, _D), lambda i, s: (0, 0)),
            pl.BlockSpec((_N, _E), lambda i, s: (0, 0)),
            pl.BlockSpec((1, _F, _D), lambda i, s: (s[i], 0, 0)),
            pl.BlockSpec((1, 1, _F), lambda i, s: (s[i], 0, 0)),
            pl.BlockSpec((1, _D, _F), lambda i, s: (s[i], 0, 0)),
            pl.BlockSpec((1, 1, _D), lambda i, s: (s[i], 0, 0)),
            pl.BlockSpec((1, _D), lambda i, s: (0, 0)),
            pl.BlockSpec((1, _D), lambda i, s: (0, 0)),
        ],
        out_specs=pl.BlockSpec((_N, _D), lambda i, s: (0, 0)),
        scratch_shapes=[
            pltpu.VMEM((_N, _D), jnp.float32),
            pltpu.VMEM((_N, _E), jnp.float32),
        ],
    )
    out = pl.pallas_call(
        _fwd,
        grid_spec=grid_spec,
        out_shape=jax.ShapeDtypeStruct((_N, _D), jnp.float32),
        compiler_params=pltpu.CompilerParams(
            dimension_semantics=("arbitrary",),
        ),
    )(used, xf, hexf, W_in, bi3, W_out, bo3, g2, bt2)
    return out.reshape(_B, _T, _D)
